# Initial kernel scaffold; baseline (speedup 1.0000x reference)
import jax
import jax.numpy as jnp
from jax.experimental import pallas as pl


def kernel(x, edge_index, weights):
    raise NotImplementedError("write your pallas kernel here")


# speedup vs baseline: 4.0880x; 4.0880x over previous
import functools

import jax
import jax.numpy as jnp
from jax import lax
from jax.experimental import pallas as pl
from jax.experimental.pallas import tpu as pltpu
from jax.experimental.pallas import tpu_sc as plsc

N = 10000
E = 320000
B = 128
T = 5
LR = 0.01
N_SENSORY = 2048

NC = 2
NSUB = 16
NW = NC * NSUB
CHUNK = 128
EW = ((E // NW) + CHUNK - 1) // CHUNK * CHUNK
NCHUNK = EW // CHUNK
EPAD = EW * NW
SUB_STRIDE = 624
SUB_ROWS = 640

_mesh = plsc.VectorSubcoreMesh(core_axis_name="c", subcore_axis_name="s")


def _sc_pass_body(tab_hbm, gidx_hbm, sidx_hbm, w_hbm, out_hbm,
                  gidx_v, sidx_v, w_v, rows_v, y_sh, sem):
    c = lax.axis_index("c")
    s = lax.axis_index("s")
    g = c * NSUB + s

    pltpu.sync_copy(gidx_hbm.at[g], gidx_v)
    pltpu.sync_copy(sidx_hbm.at[g], sidx_v)
    pltpu.sync_copy(w_hbm.at[g], w_v)

    def _zrow(j, carry):
        for r in range(B // 16):
            rows_v[j, pl.ds(r * 16, 16)] = jnp.zeros((16,), jnp.float32)
        return carry
    lax.fori_loop(0, CHUNK, _zrow, 0)
    base = s * SUB_STRIDE
    for k in range(SUB_ROWS // CHUNK):
        pltpu.sync_copy(rows_v, y_sh.at[pl.ds(base + k * CHUNK, CHUNK)])
    plsc.subcore_barrier()

    def _chunk(ci, carry):
        pltpu.async_copy(tab_hbm.at[gidx_v.at[ci]], rows_v, sem).wait()

        def _scale(j2, inner):
            wvec = w_v[ci, pl.ds(j2 * 16, 16)]
            for l in range(16):
                wj = wvec[l]
                e = j2 * 16 + l
                for r in range(B // 16):
                    rows_v[e, pl.ds(r * 16, 16)] = rows_v[e, pl.ds(r * 16, 16)] * wj
            return inner
        lax.fori_loop(0, CHUNK // 16, _scale, 0)

        pltpu.sync_copy(rows_v, y_sh.at[sidx_v.at[ci]], add=True)
        return carry
    lax.fori_loop(0, NCHUNK, _chunk, 0)
    plsc.subcore_barrier()

    pltpu.sync_copy(y_sh.at[pl.ds(base, SUB_ROWS)],
                    out_hbm.at[c, pl.ds(base, SUB_ROWS)])


_sc_pass = functools.partial(
    pl.kernel,
    out_type=jax.ShapeDtypeStruct((NC, N, B), jnp.float32),
    mesh=_mesh,
    scratch_types=[
        pltpu.VMEM((NCHUNK, CHUNK), jnp.int32),
        pltpu.VMEM((NCHUNK, CHUNK), jnp.int32),
        pltpu.VMEM((NCHUNK, CHUNK), jnp.float32),
        pltpu.VMEM((CHUNK, B), jnp.float32),
        pltpu.VMEM_SHARED((N, B), jnp.float32),
        pltpu.SemaphoreType.DMA,
    ],
)(_sc_pass_body)


_RB = 1000
_GRID = N // _RB


def _act_body(v_ref, a_ref):
    a_ref[...] = jnp.tanh(v_ref[...])


_act_call = pl.pallas_call(
    _act_body, grid=(_GRID,),
    in_specs=[pl.BlockSpec((_RB, B), lambda i: (i, 0))],
    out_specs=pl.BlockSpec((_RB, B), lambda i: (i, 0)),
    out_shape=jax.ShapeDtypeStruct((N, B), jnp.float32))


def _err_body(v_ref, p_ref, e_ref):
    e_ref[...] = v_ref[...] - p_ref[0] - p_ref[1]


_err_call = pl.pallas_call(
    _err_body, grid=(_GRID,),
    in_specs=[pl.BlockSpec((_RB, B), lambda i: (i, 0)),
              pl.BlockSpec((NC, _RB, B), lambda i: (0, i, 0))],
    out_specs=pl.BlockSpec((_RB, B), lambda i: (i, 0)),
    out_shape=jax.ShapeDtypeStruct((N, B), jnp.float32))


def _upd_body(v_ref, a_ref, e_ref, b_ref, vo_ref, ao_ref):
    act = a_ref[...]
    back = (b_ref[0] + b_ref[1]) * (1.0 - act * act)
    grad = e_ref[...] - back
    rows = pl.program_id(0) * _RB + lax.broadcasted_iota(jnp.int32, (_RB, B), 0)
    mask = (rows >= N_SENSORY).astype(jnp.float32)
    vn = v_ref[...] - LR * mask * grad
    vo_ref[...] = vn
    ao_ref[...] = jnp.tanh(vn)


_upd_call = pl.pallas_call(
    _upd_body, grid=(_GRID,),
    in_specs=[pl.BlockSpec((_RB, B), lambda i: (i, 0)),
              pl.BlockSpec((_RB, B), lambda i: (i, 0)),
              pl.BlockSpec((_RB, B), lambda i: (i, 0)),
              pl.BlockSpec((NC, _RB, B), lambda i: (0, i, 0))],
    out_specs=[pl.BlockSpec((_RB, B), lambda i: (i, 0)),
               pl.BlockSpec((_RB, B), lambda i: (i, 0))],
    out_shape=[jax.ShapeDtypeStruct((N, B), jnp.float32),
               jax.ShapeDtypeStruct((N, B), jnp.float32)])


def kernel(x, edge_index, weights):
    src = edge_index[0]
    dst = edge_index[1]
    pad = EPAD - E
    srcp = jnp.pad(src, (0, pad)).reshape(NW, NCHUNK, CHUNK)
    dstp = jnp.pad(dst, (0, pad)).reshape(NW, NCHUNK, CHUNK)
    wp = jnp.pad(weights, (0, pad)).reshape(NW, NCHUNK, CHUNK)

    values = x
    act = _act_call(values)
    for _ in range(T):
        pred = _sc_pass(act, srcp, dstp, wp)
        err = _err_call(values, pred)
        back = _sc_pass(err, dstp, srcp, wp)
        values, act = _upd_call(values, act, err, back)
    return values
